# Initial kernel scaffold; baseline (speedup 1.0000x reference)
import jax
import jax.numpy as jnp
from jax.experimental import pallas as pl


def kernel(x, edge_index, W, b):
    raise NotImplementedError("write your pallas kernel here")


# speedup vs baseline: 3.5873x; 3.5873x over previous
import jax
import jax.numpy as jnp
from jax import lax
from jax.experimental import pallas as pl
from jax.experimental.pallas import tpu as pltpu
from jax.experimental.pallas import tpu_sc as plsc

N_NODES = 10000
N_EDGES = 320000
D = 128
DH = D // 2

NC = 2
NS = 16
EDGES_PER_TILE = N_EDGES // NS
CHUNK = 80
STEPS = EDGES_PER_TILE // CHUNK
CNT_STEPS = STEPS // NC
NPAD = 12800
ROWS_PER_TILE = NPAD // NS
ZROWS = 160


def _sc_accumulate(rows_hbm, cols_hbm, xs_hbm, sum_hbm, cnt_hbm,
                   ridx_v, cidx_v, feat_v, ones_v, zrow_v, zcnt_v,
                   ssum, scnt, sem):
    c = lax.axis_index("c")
    s = lax.axis_index("s")

    def fill_z(i, carry):
        for j in range(DH // 16):
            zrow_v[i, pl.ds(j * 16, 16)] = jnp.zeros((16,), jnp.float32)
        zcnt_v[i, :] = jnp.zeros((16,), jnp.float32)
        return carry
    lax.fori_loop(0, ZROWS, fill_z, 0)

    def fill_o(i, carry):
        ones_v[i, :] = jnp.full((16,), 1.0, jnp.float32)
        return carry
    lax.fori_loop(0, CHUNK, fill_o, 0)

    rbase = s * ROWS_PER_TILE
    for k in range(ROWS_PER_TILE // ZROWS):
        pltpu.sync_copy(zrow_v, ssum.at[pl.ds(rbase + k * ZROWS, ZROWS)])
        pltpu.sync_copy(zcnt_v, scnt.at[pl.ds(rbase + k * ZROWS, ZROWS)])
    plsc.subcore_barrier()

    ebase = s * EDGES_PER_TILE
    xbase = c * N_NODES

    def step(t, carry):
        off = pl.multiple_of(ebase + t * CHUNK, 8)
        pltpu.sync_copy(rows_hbm.at[pl.ds(off, CHUNK)], ridx_v)
        pltpu.sync_copy(cols_hbm.at[pl.ds(off, CHUNK)], cidx_v)
        pltpu.async_copy(xs_hbm.at[pl.ds(xbase, N_NODES)].at[cidx_v],
                         feat_v, sem).wait()
        pltpu.sync_copy(feat_v, ssum.at[ridx_v], add=True)

        @pl.when(t // CNT_STEPS == c)
        def _count():
            pltpu.sync_copy(ones_v, scnt.at[ridx_v], add=True)
        return carry
    lax.fori_loop(0, STEPS, step, 0)

    plsc.subcore_barrier()

    obase = c * NPAD + rbase
    pltpu.sync_copy(ssum.at[pl.ds(rbase, ROWS_PER_TILE)],
                    sum_hbm.at[pl.ds(obase, ROWS_PER_TILE)])
    pltpu.sync_copy(scnt.at[pl.ds(rbase, ROWS_PER_TILE)],
                    cnt_hbm.at[pl.ds(obase, ROWS_PER_TILE)])


def _tc_dense(x_ref, slo_ref, shi_ref, c0_ref, c1_ref, w1_ref, w2a_ref,
              w2b_ref, b_ref, o_ref):
    cnt = c0_ref[...][:, 0:1] + c1_ref[...][:, 0:1]
    inv = 1.0 / (cnt + 1e-8)
    acc = jnp.dot(x_ref[...], w1_ref[...], preferred_element_type=jnp.float32)
    acc = acc + jnp.dot(slo_ref[...] * inv, w2a_ref[...],
                        preferred_element_type=jnp.float32)
    acc = acc + jnp.dot(shi_ref[...] * inv, w2b_ref[...],
                        preferred_element_type=jnp.float32)
    o_ref[...] = jnp.maximum(acc + b_ref[...], 0.0)


@jax.jit
def kernel(x, edge_index, W, b):
    ei = edge_index.astype(jnp.int32)
    rows = ei[0]
    cols = ei[1]
    xs = jnp.concatenate([x[:, :DH], x[:, DH:]], axis=0)

    mesh = plsc.VectorSubcoreMesh(core_axis_name="c", subcore_axis_name="s")
    sc = pl.kernel(
        _sc_accumulate,
        out_type=(
            jax.ShapeDtypeStruct((NC * NPAD, DH), jnp.float32),
            jax.ShapeDtypeStruct((NC * NPAD, 16), jnp.float32),
        ),
        mesh=mesh,
        scratch_types=[
            pltpu.VMEM((CHUNK,), jnp.int32),
            pltpu.VMEM((CHUNK,), jnp.int32),
            pltpu.VMEM((CHUNK, DH), jnp.float32),
            pltpu.VMEM((CHUNK, 16), jnp.float32),
            pltpu.VMEM((ZROWS, DH), jnp.float32),
            pltpu.VMEM((ZROWS, 16), jnp.float32),
            pltpu.VMEM_SHARED((NPAD, DH), jnp.float32),
            pltpu.VMEM_SHARED((NPAD, 16), jnp.float32),
            pltpu.SemaphoreType.DMA,
        ],
        compiler_params=pltpu.CompilerParams(use_tc_tiling_on_sc=False),
    )
    sum_p, cnt_p = sc(rows, cols, xs)

    wt = W.T
    w1 = wt[:D]
    w2a = wt[D:D + DH]
    w2b = wt[D + DH:]
    b2 = b.reshape(1, -1)

    blk = 400
    nblk = NPAD // blk
    out = pl.pallas_call(
        _tc_dense,
        grid=(N_NODES // blk,),
        in_specs=[
            pl.BlockSpec((blk, D), lambda i: (i, 0)),
            pl.BlockSpec((blk, DH), lambda i: (i, 0)),
            pl.BlockSpec((blk, DH), lambda i: (i + nblk, 0)),
            pl.BlockSpec((blk, 16), lambda i: (i, 0)),
            pl.BlockSpec((blk, 16), lambda i: (i + nblk, 0)),
            pl.BlockSpec((D, D), lambda i: (0, 0)),
            pl.BlockSpec((DH, D), lambda i: (0, 0)),
            pl.BlockSpec((DH, D), lambda i: (0, 0)),
            pl.BlockSpec((1, D), lambda i: (0, 0)),
        ],
        out_specs=pl.BlockSpec((blk, D), lambda i: (i, 0)),
        out_shape=jax.ShapeDtypeStruct((N_NODES, D), jnp.float32),
    )(x, sum_p, sum_p, cnt_p, cnt_p, w1, w2a, w2b, b2)
    return out


# speedup vs baseline: 7.4431x; 2.0749x over previous
import jax
import jax.numpy as jnp
from jax import lax
from jax.experimental import pallas as pl
from jax.experimental.pallas import tpu as pltpu
from jax.experimental.pallas import tpu_sc as plsc

N_NODES = 10000
N_EDGES = 320000
D = 128
DH = D // 2

NC = 2
NS = 16
EDGES_PER_TILE = N_EDGES // NS
CHUNK = 125
STEPS = EDGES_PER_TILE // CHUNK
CNT_STEPS = STEPS // NC
NPAD = 10240
ROWS_PER_TILE = NPAD // NS
ZROWS = 160


def _sc_accumulate(rows_hbm, cols_hbm, xs_hbm, sum_hbm, cnt_hbm,
                   ridx_v, cidx_v, feat0, feat1, ones_v, zrow_v, zcnt_v,
                   ssum, scnt, sem0, sem1):
    c = lax.axis_index("c")
    s = lax.axis_index("s")

    def fill_z(i, carry):
        for j in range(DH // 16):
            zrow_v[i, pl.ds(j * 16, 16)] = jnp.zeros((16,), jnp.float32)
        zcnt_v[i, :] = jnp.zeros((16,), jnp.float32)
        return carry
    lax.fori_loop(0, ZROWS, fill_z, 0)

    def fill_o(i, carry):
        ones_v[i, :] = jnp.full((16,), 1.0, jnp.float32)
        return carry
    lax.fori_loop(0, CHUNK, fill_o, 0)

    pltpu.sync_copy(rows_hbm.at[s], ridx_v)
    pltpu.sync_copy(cols_hbm.at[s], cidx_v)

    rbase = s * ROWS_PER_TILE
    for k in range(ROWS_PER_TILE // ZROWS):
        pltpu.sync_copy(zrow_v, ssum.at[pl.ds(rbase + k * ZROWS, ZROWS)])
        pltpu.sync_copy(zcnt_v, scnt.at[pl.ds(rbase + k * ZROWS, ZROWS)])
    plsc.subcore_barrier()

    xbase = c * N_NODES
    xtab = xs_hbm.at[pl.ds(xbase, N_NODES)]
    bufs = (feat0, feat1)
    sems = (sem0, sem1)

    pltpu.async_copy(xtab.at[cidx_v.at[0]], feat0, sem0)
    pltpu.async_copy(xtab.at[cidx_v.at[1]], feat1, sem1)

    def step(i, carry):
        for k in range(2):
            t = 2 * i + k
            buf, sem = bufs[k], sems[k]
            pltpu.make_async_copy(xtab.at[pl.ds(0, CHUNK)], buf, sem).wait()
            pltpu.sync_copy(buf, ssum.at[ridx_v.at[t]], add=True)

            @pl.when(t // CNT_STEPS == c)
            def _count():
                pltpu.sync_copy(ones_v, scnt.at[ridx_v.at[t]], add=True)

            @pl.when(t + 2 < STEPS)
            def _prefetch():
                pltpu.async_copy(xtab.at[cidx_v.at[t + 2]], buf, sem)
        return carry
    lax.fori_loop(0, STEPS // 2, step, 0)

    plsc.subcore_barrier()

    obase = c * NPAD + rbase
    pltpu.sync_copy(ssum.at[pl.ds(rbase, ROWS_PER_TILE)],
                    sum_hbm.at[pl.ds(obase, ROWS_PER_TILE)])
    pltpu.sync_copy(scnt.at[pl.ds(rbase, ROWS_PER_TILE)],
                    cnt_hbm.at[pl.ds(obase, ROWS_PER_TILE)])


def _tc_dense(x_ref, slo_ref, shi_ref, c0_ref, c1_ref, w1_ref, w2a_ref,
              w2b_ref, b_ref, o_ref):
    cnt = c0_ref[...][:, 0:1] + c1_ref[...][:, 0:1]
    inv = 1.0 / (cnt + 1e-8)
    acc = jnp.dot(x_ref[...], w1_ref[...], preferred_element_type=jnp.float32)
    acc = acc + jnp.dot(slo_ref[...] * inv, w2a_ref[...],
                        preferred_element_type=jnp.float32)
    acc = acc + jnp.dot(shi_ref[...] * inv, w2b_ref[...],
                        preferred_element_type=jnp.float32)
    o_ref[...] = jnp.maximum(acc + b_ref[...], 0.0)


@jax.jit
def kernel(x, edge_index, W, b):
    ei = edge_index.astype(jnp.int32)
    rows = ei[0].reshape(NS, STEPS, CHUNK)
    cols = ei[1].reshape(NS, STEPS, CHUNK)
    xs = jnp.concatenate([x[:, :DH], x[:, DH:]], axis=0)

    mesh = plsc.VectorSubcoreMesh(core_axis_name="c", subcore_axis_name="s")
    sc = pl.kernel(
        _sc_accumulate,
        out_type=(
            jax.ShapeDtypeStruct((NC * NPAD, DH), jnp.float32),
            jax.ShapeDtypeStruct((NC * NPAD, 16), jnp.float32),
        ),
        mesh=mesh,
        scratch_types=[
            pltpu.VMEM((STEPS, CHUNK), jnp.int32),
            pltpu.VMEM((STEPS, CHUNK), jnp.int32),
            pltpu.VMEM((CHUNK, DH), jnp.float32),
            pltpu.VMEM((CHUNK, DH), jnp.float32),
            pltpu.VMEM((CHUNK, 16), jnp.float32),
            pltpu.VMEM((ZROWS, DH), jnp.float32),
            pltpu.VMEM((ZROWS, 16), jnp.float32),
            pltpu.VMEM_SHARED((NPAD, DH), jnp.float32),
            pltpu.VMEM_SHARED((NPAD, 16), jnp.float32),
            pltpu.SemaphoreType.DMA,
            pltpu.SemaphoreType.DMA,
        ],
        compiler_params=pltpu.CompilerParams(use_tc_tiling_on_sc=False),
    )
    sum_p, cnt_p = sc(rows, cols, xs)

    wt = W.T
    w1 = wt[:D]
    w2a = wt[D:D + DH]
    w2b = wt[D + DH:]
    b2 = b.reshape(1, -1)

    blk = 80
    nblk = NPAD // blk
    out = pl.pallas_call(
        _tc_dense,
        grid=(N_NODES // blk,),
        in_specs=[
            pl.BlockSpec((blk, D), lambda i: (i, 0)),
            pl.BlockSpec((blk, DH), lambda i: (i, 0)),
            pl.BlockSpec((blk, DH), lambda i: (i + nblk, 0)),
            pl.BlockSpec((blk, 16), lambda i: (i, 0)),
            pl.BlockSpec((blk, 16), lambda i: (i + nblk, 0)),
            pl.BlockSpec((D, D), lambda i: (0, 0)),
            pl.BlockSpec((DH, D), lambda i: (0, 0)),
            pl.BlockSpec((DH, D), lambda i: (0, 0)),
            pl.BlockSpec((1, D), lambda i: (0, 0)),
        ],
        out_specs=pl.BlockSpec((blk, D), lambda i: (i, 0)),
        out_shape=jax.ShapeDtypeStruct((N_NODES, D), jnp.float32),
    )(x, sum_p, sum_p, cnt_p, cnt_p, w1, w2a, w2b, b2)
    return out


# speedup vs baseline: 9.5828x; 1.2875x over previous
import jax
import jax.numpy as jnp
from jax import lax
from jax.experimental import pallas as pl
from jax.experimental.pallas import tpu as pltpu
from jax.experimental.pallas import tpu_sc as plsc

N_NODES = 10000
N_EDGES = 320000
D = 128
DH = D // 2

NC = 2
NS = 16
EDGES_PER_TILE = N_EDGES // NS
CHUNK = 125
STEPS = EDGES_PER_TILE // CHUNK
CNT_STEPS = STEPS // NC
NPAD = 10240
ROWS_PER_TILE = NPAD // NS
ZROWS = 160


def _sc_accumulate(rows_hbm, cols_hbm, xs_hbm, slo_hbm, shi_hbm,
                   cnt0_hbm, cnt1_hbm,
                   ridx_v, cidx_v, feat0, feat1, ones_v, zrow_v, zcnt_v,
                   ssum, scnt, sem0, sem1):
    c = lax.axis_index("c")
    s = lax.axis_index("s")

    def fill_z(i, carry):
        for j in range(DH // 16):
            zrow_v[i, pl.ds(j * 16, 16)] = jnp.zeros((16,), jnp.float32)
        zcnt_v[i, :] = jnp.zeros((16,), jnp.float32)
        return carry
    lax.fori_loop(0, ZROWS, fill_z, 0)

    def fill_o(i, carry):
        ones_v[i, :] = jnp.full((16,), 1.0, jnp.float32)
        return carry
    lax.fori_loop(0, CHUNK, fill_o, 0)

    pltpu.sync_copy(rows_hbm.at[s], ridx_v)
    pltpu.sync_copy(cols_hbm.at[s], cidx_v)

    rbase = s * ROWS_PER_TILE
    for k in range(ROWS_PER_TILE // ZROWS):
        pltpu.sync_copy(zrow_v, ssum.at[pl.ds(rbase + k * ZROWS, ZROWS)])
        pltpu.sync_copy(zcnt_v, scnt.at[pl.ds(rbase + k * ZROWS, ZROWS)])
    plsc.subcore_barrier()

    xbase = c * N_NODES
    xtab = xs_hbm.at[pl.ds(xbase, N_NODES)]
    bufs = (feat0, feat1)
    sems = (sem0, sem1)

    pltpu.async_copy(xtab.at[cidx_v.at[0]], feat0, sem0)
    pltpu.async_copy(xtab.at[cidx_v.at[1]], feat1, sem1)

    def step(i, carry):
        for k in range(2):
            t = 2 * i + k
            buf, sem = bufs[k], sems[k]
            pltpu.make_async_copy(xtab.at[pl.ds(0, CHUNK)], buf, sem).wait()
            pltpu.sync_copy(buf, ssum.at[ridx_v.at[t]], add=True)

            @pl.when(t // CNT_STEPS == c)
            def _count():
                pltpu.sync_copy(ones_v, scnt.at[ridx_v.at[t]], add=True)

            @pl.when(t + 2 < STEPS)
            def _prefetch():
                pltpu.async_copy(xtab.at[cidx_v.at[t + 2]], buf, sem)
        return carry
    lax.fori_loop(0, STEPS // 2, step, 0)

    plsc.subcore_barrier()

    @pl.when(c == 0)
    def _out0():
        pltpu.sync_copy(ssum.at[pl.ds(rbase, ROWS_PER_TILE)],
                        slo_hbm.at[pl.ds(rbase, ROWS_PER_TILE)])
        pltpu.sync_copy(scnt.at[pl.ds(rbase, ROWS_PER_TILE)],
                        cnt0_hbm.at[pl.ds(rbase, ROWS_PER_TILE)])

    @pl.when(c == 1)
    def _out1():
        pltpu.sync_copy(ssum.at[pl.ds(rbase, ROWS_PER_TILE)],
                        shi_hbm.at[pl.ds(rbase, ROWS_PER_TILE)])
        pltpu.sync_copy(scnt.at[pl.ds(rbase, ROWS_PER_TILE)],
                        cnt1_hbm.at[pl.ds(rbase, ROWS_PER_TILE)])


def _tc_dense(x_ref, slo_ref, shi_ref, c0_ref, c1_ref, w1_ref, w2a_ref,
              w2b_ref, b_ref, o_ref):
    cnt = c0_ref[...][:, 0:1] + c1_ref[...][:, 0:1]
    inv = 1.0 / (cnt + 1e-8)
    acc = jnp.dot(x_ref[...], w1_ref[...], preferred_element_type=jnp.float32)
    acc = acc + jnp.dot(slo_ref[...] * inv, w2a_ref[...],
                        preferred_element_type=jnp.float32)
    acc = acc + jnp.dot(shi_ref[...] * inv, w2b_ref[...],
                        preferred_element_type=jnp.float32)
    o_ref[...] = jnp.maximum(acc + b_ref[...], 0.0)


@jax.jit
def kernel(x, edge_index, W, b):
    ei = edge_index.astype(jnp.int32)
    rows = ei[0].reshape(NS, STEPS, CHUNK)
    cols = ei[1].reshape(NS, STEPS, CHUNK)
    xs = jnp.concatenate([x[:, :DH], x[:, DH:]], axis=0)

    mesh = plsc.VectorSubcoreMesh(core_axis_name="c", subcore_axis_name="s")
    sc = pl.kernel(
        _sc_accumulate,
        out_type=(
            jax.ShapeDtypeStruct((NPAD, DH), jnp.float32),
            jax.ShapeDtypeStruct((NPAD, DH), jnp.float32),
            jax.ShapeDtypeStruct((NPAD, 16), jnp.float32),
            jax.ShapeDtypeStruct((NPAD, 16), jnp.float32),
        ),
        mesh=mesh,
        scratch_types=[
            pltpu.VMEM((STEPS, CHUNK), jnp.int32),
            pltpu.VMEM((STEPS, CHUNK), jnp.int32),
            pltpu.VMEM((CHUNK, DH), jnp.float32),
            pltpu.VMEM((CHUNK, DH), jnp.float32),
            pltpu.VMEM((CHUNK, 16), jnp.float32),
            pltpu.VMEM((ZROWS, DH), jnp.float32),
            pltpu.VMEM((ZROWS, 16), jnp.float32),
            pltpu.VMEM_SHARED((NPAD, DH), jnp.float32),
            pltpu.VMEM_SHARED((NPAD, 16), jnp.float32),
            pltpu.SemaphoreType.DMA,
            pltpu.SemaphoreType.DMA,
        ],
        compiler_params=pltpu.CompilerParams(use_tc_tiling_on_sc=False),
    )
    s_lo, s_hi, cnt0, cnt1 = sc(rows, cols, xs)

    wt = W.T
    w1 = wt[:D]
    w2a = wt[D:D + DH]
    w2b = wt[D + DH:]
    b2 = b.reshape(1, -1)

    blk = 1000
    out = pl.pallas_call(
        _tc_dense,
        grid=(N_NODES // blk,),
        in_specs=[
            pl.BlockSpec((blk, D), lambda i: (i, 0)),
            pl.BlockSpec((blk, DH), lambda i: (i, 0)),
            pl.BlockSpec((blk, DH), lambda i: (i, 0)),
            pl.BlockSpec((blk, 16), lambda i: (i, 0)),
            pl.BlockSpec((blk, 16), lambda i: (i, 0)),
            pl.BlockSpec((D, D), lambda i: (0, 0)),
            pl.BlockSpec((DH, D), lambda i: (0, 0)),
            pl.BlockSpec((DH, D), lambda i: (0, 0)),
            pl.BlockSpec((1, D), lambda i: (0, 0)),
        ],
        out_specs=pl.BlockSpec((blk, D), lambda i: (i, 0)),
        out_shape=jax.ShapeDtypeStruct((N_NODES, D), jnp.float32),
    )(x, s_lo, s_hi, cnt0, cnt1, w1, w2a, w2b, b2)
    return out
